# baseline (speedup 1.0000x reference)
import functools

import jax
import jax.numpy as jnp
from jax import lax
from jax.experimental import pallas as pl
from jax.experimental.pallas import tpu as pltpu
from jax.experimental.pallas import tpu_sc as plsc

N = 10000
E = 160000
D_IN = 256
D_H = 128
VIS = 256

NC = 2
NS = 16
L = 16
NW = NC * NS
EW = E // NW
RB = 2000
GRID = N // RB

CH_C = 32
NCH_C = EW // CH_C
REM_C = EW - NCH_C * CH_C

CH_V = 64
NCH_V = EW // CH_V
REM_V = EW - NCH_V * CH_V

ROWS_T = N // NS
ZR = 125

_mesh = plsc.VectorSubcoreMesh(core_axis_name="c", subcore_axis_name="s",
                               num_cores=NC, num_subcores=NS)


def _tc1_body(x_ref, vis_ref, wm_ref, bm_ref, wc1_ref, bc1_ref, wnp_ref,
              bnp_ref, bcp_ref, m1_ref, visn_ref, ns_ref):
    h = jnp.dot(x_ref[...], wm_ref[...], preferred_element_type=jnp.float32)
    h = h + bm_ref[...]
    m1_ref[...] = jnp.dot(h, wc1_ref[...],
                          preferred_element_type=jnp.float32) + bc1_ref[...]
    v = vis_ref[...]
    nrm = jnp.sqrt(jnp.sum(v * v, axis=1, keepdims=True)) + 1e-12
    visn_ref[...] = v / nrm
    ns = jnp.sum(h * wnp_ref[...], axis=1) + bnp_ref[0, 0] + bcp_ref[0, 0]
    ns_ref[...] = ns[:, None]


def _tc1(x, vis, W_mlp, b_mlp, W_c1, b_c1, wnp, bnp, bcp):
    full = lambda s: pl.BlockSpec(s, lambda i: (0, 0))
    return pl.pallas_call(
        _tc1_body,
        grid=(GRID,),
        in_specs=[
            pl.BlockSpec((RB, D_IN), lambda i: (i, 0)),
            pl.BlockSpec((RB, VIS), lambda i: (i, 0)),
            full((D_IN, D_H)), full((1, D_H)),
            full((D_H, D_H)), full((1, D_H)),
            full((1, D_H)), full((1, 1)), full((1, 1)),
        ],
        out_specs=[
            pl.BlockSpec((RB, D_H), lambda i: (i, 0)),
            pl.BlockSpec((RB, VIS), lambda i: (i, 0)),
            pl.BlockSpec((RB, 1), lambda i: (i, 0)),
        ],
        out_shape=[
            jax.ShapeDtypeStruct((N, D_H), jnp.float32),
            jax.ShapeDtypeStruct((N, VIS), jnp.float32),
            jax.ShapeDtypeStruct((N, 1), jnp.float32),
        ],
    )(x, vis, W_mlp, b_mlp, W_c1, b_c1, wnp, bnp, bcp)


def _cosw_body(visn_hbm, src_hbm, dst_hbm, cosw_hbm,
               sidx, didx, srows0, drows0, srows1, drows1, cw, tbuf,
               ss0, sd0, ss1, sd1):
    wid = lax.axis_index("c") * NS + lax.axis_index("s")
    base = wid * EW
    ci = pltpu.async_copy(src_hbm.at[pl.ds(base, EW)], sidx, ss0)
    cj = pltpu.async_copy(dst_hbm.at[pl.ds(base, EW)], didx, sd0)
    ci.wait()
    cj.wait()

    lane = lax.iota(jnp.int32, L)

    def start(e0, n, sbuf, dbuf, sems, semd):
        pltpu.async_copy(visn_hbm.at[sidx.at[pl.ds(e0, n)]],
                         sbuf.at[pl.ds(0, n)], sems)
        pltpu.async_copy(visn_hbm.at[didx.at[pl.ds(e0, n)]],
                         dbuf.at[pl.ds(0, n)], semd)

    def wait(n, sbuf, dbuf, sems, semd):
        pltpu.make_async_copy(visn_hbm.at[sidx.at[pl.ds(0, n)]],
                              sbuf.at[pl.ds(0, n)], sems).wait()
        pltpu.make_async_copy(visn_hbm.at[didx.at[pl.ds(0, n)]],
                              dbuf.at[pl.ds(0, n)], semd).wait()

    def compute(e0, n, sbuf, dbuf):
        KC = VIS // L
        for g in range((n + L - 1) // L):
            m = min(L, n - g * L)
            for jj in range(m):
                j = g * L + jj
                accs = [sbuf[j, pl.ds(q * L, L)] * dbuf[j, pl.ds(q * L, L)]
                        for q in range(4)]
                for k in range(4, KC, 4):
                    for q in range(4):
                        accs[q] = accs[q] + (sbuf[j, pl.ds((k + q) * L, L)]
                                             * dbuf[j, pl.ds((k + q) * L, L)])
                tbuf[jj] = (accs[0] + accs[1]) + (accs[2] + accs[3])
            rs = plsc.load_gather(tbuf, [lane, jnp.zeros((L,), jnp.int32)])
            for k in range(1, L):
                rs = rs + plsc.load_gather(tbuf, [lane, jnp.full((L,), k, jnp.int32)])
            cw[pl.ds(e0 + g * L, L)] = rs

    start(0, CH_C, srows0, drows0, ss0, sd0)

    @pl.loop(0, NCH_C, step=2)
    def _chunks(c):
        e0 = c * CH_C
        start(e0 + CH_C, CH_C, srows1, drows1, ss1, sd1)
        wait(CH_C, srows0, drows0, ss0, sd0)
        compute(e0, CH_C, srows0, drows0)

        @pl.when(c + 2 < NCH_C)
        def _pref():
            start(e0 + 2 * CH_C, CH_C, srows0, drows0, ss0, sd0)

        wait(CH_C, srows1, drows1, ss1, sd1)
        compute(e0 + CH_C, CH_C, srows1, drows1)

    if REM_C:
        start(NCH_C * CH_C, REM_C, srows0, drows0, ss0, sd0)
        wait(REM_C, srows0, drows0, ss0, sd0)
        compute(NCH_C * CH_C, REM_C, srows0, drows0)

    pltpu.sync_copy(cw.at[pl.ds(0, EW)], cosw_hbm.at[pl.ds(base, EW)])


def _cosw(visn, src, dst):
    f = functools.partial(
        pl.kernel,
        out_type=jax.ShapeDtypeStruct((E,), jnp.float32),
        mesh=_mesh,
        compiler_params=pltpu.CompilerParams(needs_layout_passes=False, use_tc_tiling_on_sc=False),
        scratch_types=[
            pltpu.MemorySpace.VMEM((EW,), jnp.int32),
            pltpu.MemorySpace.VMEM((EW,), jnp.int32),
            pltpu.MemorySpace.VMEM((CH_C, VIS), jnp.float32),
            pltpu.MemorySpace.VMEM((CH_C, VIS), jnp.float32),
            pltpu.MemorySpace.VMEM((CH_C, VIS), jnp.float32),
            pltpu.MemorySpace.VMEM((CH_C, VIS), jnp.float32),
            pltpu.MemorySpace.VMEM((EW + L,), jnp.float32),
            pltpu.MemorySpace.VMEM((L, L), jnp.float32),
            pltpu.SemaphoreType.DMA,
            pltpu.SemaphoreType.DMA,
            pltpu.SemaphoreType.DMA,
            pltpu.SemaphoreType.DMA,
        ],
    )(_cosw_body)
    return f(visn, src, dst)


def _conv_body(m_hbm, src_hbm, dst_hbm, cosw_hbm, out_hbm,
               acc_sh, sidx, didx, cwv, rows0, rows1, zbuf,
               sst, sg0, sg1):
    cid = lax.axis_index("c")
    sid = lax.axis_index("s")
    wid = cid * NS + sid
    base = wid * EW

    c1 = pltpu.async_copy(src_hbm.at[pl.ds(base, EW)], sidx, sst)
    c2 = pltpu.async_copy(dst_hbm.at[pl.ds(base, EW)], didx, sst)
    c3 = pltpu.async_copy(cosw_hbm.at[pl.ds(base, EW)], cwv.at[pl.ds(0, EW)], sst)

    @pl.loop(0, ZR)
    def _zb(r):
        for k in range(D_H // L):
            zbuf[r, pl.ds(k * L, L)] = jnp.zeros((L,), jnp.float32)

    c1.wait()
    c2.wait()
    c3.wait()

    for i in range(ROWS_T // ZR):
        pltpu.async_copy(zbuf, acc_sh.at[pl.ds(sid * ROWS_T + i * ZR, ZR)], sst)
    for i in range(ROWS_T // ZR):
        pltpu.make_async_copy(zbuf, acc_sh.at[pl.ds(sid * ROWS_T, ZR)], sst).wait()
    plsc.subcore_barrier()

    def start(e0, n, buf, sem):
        pltpu.async_copy(m_hbm.at[sidx.at[pl.ds(e0, n)]],
                         buf.at[pl.ds(0, n)], sem)

    def wait(n, buf, sem):
        pltpu.make_async_copy(m_hbm.at[sidx.at[pl.ds(0, n)]],
                              buf.at[pl.ds(0, n)], sem).wait()

    def scale_scatter(e0, n, buf):
        for g in range((n + L - 1) // L):
            m = min(L, n - g * L)
            wv = cwv[pl.ds(e0 + g * L, L)]
            for jj in range(m):
                j = g * L + jj
                w = wv[jj]
                for k in range(D_H // L):
                    buf[j, pl.ds(k * L, L)] = buf[j, pl.ds(k * L, L)] * w
        pltpu.sync_copy(buf.at[pl.ds(0, n)],
                        acc_sh.at[didx.at[pl.ds(e0, n)]], add=True)

    start(0, CH_V, rows0, sg0)

    @pl.loop(0, NCH_V, step=2)
    def _chunks(c):
        e0 = c * CH_V
        start(e0 + CH_V, CH_V, rows1, sg1)
        wait(CH_V, rows0, sg0)
        scale_scatter(e0, CH_V, rows0)

        @pl.when(c + 2 < NCH_V)
        def _pref():
            start(e0 + 2 * CH_V, CH_V, rows0, sg0)

        wait(CH_V, rows1, sg1)
        scale_scatter(e0 + CH_V, CH_V, rows1)

    if REM_V:
        start(NCH_V * CH_V, REM_V, rows0, sg0)
        wait(REM_V, rows0, sg0)
        scale_scatter(NCH_V * CH_V, REM_V, rows0)

    plsc.subcore_barrier()
    pltpu.sync_copy(acc_sh.at[pl.ds(sid * ROWS_T, ROWS_T)],
                    out_hbm.at[cid, pl.ds(sid * ROWS_T, ROWS_T)])


def _conv(m, src, dst, cosw):
    f = functools.partial(
        pl.kernel,
        out_type=jax.ShapeDtypeStruct((NC, N, D_H), jnp.float32),
        mesh=_mesh,
        compiler_params=pltpu.CompilerParams(needs_layout_passes=False, use_tc_tiling_on_sc=False),
        scratch_types=[
            pltpu.MemorySpace.VMEM_SHARED((N, D_H), jnp.float32),
            pltpu.MemorySpace.VMEM((EW,), jnp.int32),
            pltpu.MemorySpace.VMEM((EW,), jnp.int32),
            pltpu.MemorySpace.VMEM((EW + L,), jnp.float32),
            pltpu.MemorySpace.VMEM((CH_V, D_H), jnp.float32),
            pltpu.MemorySpace.VMEM((CH_V, D_H), jnp.float32),
            pltpu.MemorySpace.VMEM((ZR, D_H), jnp.float32),
            pltpu.SemaphoreType.DMA,
            pltpu.SemaphoreType.DMA,
            pltpu.SemaphoreType.DMA,
        ],
    )(_conv_body)
    return f(m, src, dst, cosw)


def _tc2_body(p_ref, wc2_ref, bc2_ref, m2_ref):
    h1 = jnp.maximum(p_ref[0] + p_ref[1], 0.0)
    m2_ref[...] = jnp.dot(h1, wc2_ref[...],
                          preferred_element_type=jnp.float32) + bc2_ref[...]


def _tc2(p, W_c2, b_c2):
    return pl.pallas_call(
        _tc2_body,
        grid=(GRID,),
        in_specs=[
            pl.BlockSpec((NC, RB, D_H), lambda i: (0, i, 0)),
            pl.BlockSpec((D_H, D_H), lambda i: (0, 0)),
            pl.BlockSpec((1, D_H), lambda i: (0, 0)),
        ],
        out_specs=pl.BlockSpec((RB, D_H), lambda i: (i, 0)),
        out_shape=jax.ShapeDtypeStruct((N, D_H), jnp.float32),
    )(p, W_c2, b_c2)


def _tc3_body(p_ref, ns_ref, wcp_ref, out_ref):
    h2 = jnp.maximum(p_ref[0] + p_ref[1], 0.0)
    out_ref[...] = ns_ref[...] + jnp.sum(h2 * wcp_ref[...], axis=1)[:, None]


def _tc3(p, ns, wcp):
    return pl.pallas_call(
        _tc3_body,
        grid=(GRID,),
        in_specs=[
            pl.BlockSpec((NC, RB, D_H), lambda i: (0, i, 0)),
            pl.BlockSpec((RB, 1), lambda i: (i, 0)),
            pl.BlockSpec((1, D_H), lambda i: (0, 0)),
        ],
        out_specs=pl.BlockSpec((RB, 1), lambda i: (i, 0)),
        out_shape=jax.ShapeDtypeStruct((N, 1), jnp.float32),
    )(p, ns, wcp)


def kernel(x, edge_index, vis, W_mlp, b_mlp, W_np, b_np, W_cp, b_cp,
           W_c1, b_c1, W_c2, b_c2):
    src = edge_index[0]
    dst = edge_index[1]
    wnp = W_np.reshape(1, D_H)
    wcp = W_cp.reshape(1, D_H)
    bnp = b_np.reshape(1, 1)
    bcp = b_cp.reshape(1, 1)

    m1, visn, ns = _tc1(x, vis, W_mlp, b_mlp.reshape(1, D_H),
                        W_c1, b_c1.reshape(1, D_H), wnp, bnp, bcp)
    cosw = _cosw(visn, src, dst)
    p1 = _conv(m1, src, dst, cosw)
    m2 = _tc2(p1, W_c2, b_c2.reshape(1, D_H))
    p2 = _conv(m2, src, dst, cosw)
    out = _tc3(p2, ns, wcp)
    return out.reshape(N)


# speedup vs baseline: 2.0957x; 2.0957x over previous
import functools

import jax
import jax.numpy as jnp
from jax import lax
from jax.experimental import pallas as pl
from jax.experimental.pallas import tpu as pltpu
from jax.experimental.pallas import tpu_sc as plsc

N = 10000
E = 160000
D_IN = 256
D_H = 128
VIS = 256

NC = 2
NS = 16
L = 16
NW = NC * NS
EW = E // NW
RB = 2000
GRID = N // RB

CH_C = 32
NCH_C = EW // CH_C
REM_C = EW - NCH_C * CH_C

CH_V = 64
NCH_V = EW // CH_V
REM_V = EW - NCH_V * CH_V

ROWS_T = N // NS
ZR = 125

_mesh = plsc.VectorSubcoreMesh(core_axis_name="c", subcore_axis_name="s",
                               num_cores=NC, num_subcores=NS)


def _tc1_body(x_ref, vis_ref, wm_ref, bm_ref, wc1_ref, bc1_ref, wnp_ref,
              bnp_ref, bcp_ref, m1_ref, visn_ref, ns_ref):
    h = jnp.dot(x_ref[...], wm_ref[...], preferred_element_type=jnp.float32)
    h = h + bm_ref[...]
    m1_ref[...] = jnp.dot(h, wc1_ref[...],
                          preferred_element_type=jnp.float32) + bc1_ref[...]
    v = vis_ref[...]
    nrm = jnp.sqrt(jnp.sum(v * v, axis=1, keepdims=True)) + 1e-12
    visn_ref[...] = (v / nrm).astype(jnp.bfloat16)
    ns = jnp.sum(h * wnp_ref[...], axis=1) + bnp_ref[0, 0] + bcp_ref[0, 0]
    ns_ref[...] = ns[:, None]


def _tc1(x, vis, W_mlp, b_mlp, W_c1, b_c1, wnp, bnp, bcp):
    full = lambda s: pl.BlockSpec(s, lambda i: (0, 0))
    return pl.pallas_call(
        _tc1_body,
        grid=(GRID,),
        in_specs=[
            pl.BlockSpec((RB, D_IN), lambda i: (i, 0)),
            pl.BlockSpec((RB, VIS), lambda i: (i, 0)),
            full((D_IN, D_H)), full((1, D_H)),
            full((D_H, D_H)), full((1, D_H)),
            full((1, D_H)), full((1, 1)), full((1, 1)),
        ],
        out_specs=[
            pl.BlockSpec((RB, D_H), lambda i: (i, 0)),
            pl.BlockSpec((RB, VIS), lambda i: (i, 0)),
            pl.BlockSpec((RB, 1), lambda i: (i, 0)),
        ],
        out_shape=[
            jax.ShapeDtypeStruct((N, D_H), jnp.float32),
            jax.ShapeDtypeStruct((N, VIS), jnp.bfloat16),
            jax.ShapeDtypeStruct((N, 1), jnp.float32),
        ],
    )(x, vis, W_mlp, b_mlp, W_c1, b_c1, wnp, bnp, bcp)


def _cosw_body(visn_hbm, src_hbm, dst_hbm, cosw_hbm,
               sidx, didx, srows0, drows0, srows1, drows1, cw, tbuf,
               ss0, sd0, ss1, sd1):
    wid = lax.axis_index("c") * NS + lax.axis_index("s")
    base = wid * EW
    ci = pltpu.async_copy(src_hbm.at[pl.ds(base, EW)], sidx, ss0)
    cj = pltpu.async_copy(dst_hbm.at[pl.ds(base, EW)], didx, sd0)
    ci.wait()
    cj.wait()

    lane = lax.iota(jnp.int32, L)

    def start(e0, n, sbuf, dbuf, sems, semd):
        pltpu.async_copy(visn_hbm.at[sidx.at[pl.ds(e0, n)]],
                         sbuf.at[pl.ds(0, n)], sems)
        pltpu.async_copy(visn_hbm.at[didx.at[pl.ds(e0, n)]],
                         dbuf.at[pl.ds(0, n)], semd)

    def wait(n, sbuf, dbuf, sems, semd):
        pltpu.make_async_copy(visn_hbm.at[sidx.at[pl.ds(0, n)]],
                              sbuf.at[pl.ds(0, n)], sems).wait()
        pltpu.make_async_copy(visn_hbm.at[didx.at[pl.ds(0, n)]],
                              dbuf.at[pl.ds(0, n)], semd).wait()

    def compute(e0, n, sbuf, dbuf):
        B2 = 2 * L
        NB = VIS // B2
        hmask = jnp.full((L,), jnp.int32(-65536))
        for g in range((n + L - 1) // L):
            m = min(L, n - g * L)
            for jj in range(m):
                j = g * L + jj
                accs = [None, None, None, None]
                for b in range(NB):
                    sv = sbuf[j, pl.ds(b * B2, B2)]
                    dv = dbuf[j, pl.ds(b * B2, B2)]
                    p = sv * dv
                    pi = plsc.bitcast(p, jnp.int32)
                    pe = plsc.bitcast(pi << 16, jnp.float32)
                    po = plsc.bitcast(pi & hmask, jnp.float32)
                    q = 2 * b
                    for (pp, i) in ((pe, q % 4), (po, (q + 1) % 4)):
                        accs[i] = pp if accs[i] is None else accs[i] + pp
                tbuf[jj] = (accs[0] + accs[1]) + (accs[2] + accs[3])
            rs = plsc.load_gather(tbuf, [lane, jnp.zeros((L,), jnp.int32)])
            for k in range(1, L):
                rs = rs + plsc.load_gather(tbuf, [lane, jnp.full((L,), k, jnp.int32)])
            cw[pl.ds(e0 + g * L, L)] = rs

    start(0, CH_C, srows0, drows0, ss0, sd0)

    @pl.loop(0, NCH_C, step=2)
    def _chunks(c):
        e0 = c * CH_C
        start(e0 + CH_C, CH_C, srows1, drows1, ss1, sd1)
        wait(CH_C, srows0, drows0, ss0, sd0)
        compute(e0, CH_C, srows0, drows0)

        @pl.when(c + 2 < NCH_C)
        def _pref():
            start(e0 + 2 * CH_C, CH_C, srows0, drows0, ss0, sd0)

        wait(CH_C, srows1, drows1, ss1, sd1)
        compute(e0 + CH_C, CH_C, srows1, drows1)

    if REM_C:
        start(NCH_C * CH_C, REM_C, srows0, drows0, ss0, sd0)
        wait(REM_C, srows0, drows0, ss0, sd0)
        compute(NCH_C * CH_C, REM_C, srows0, drows0)

    pltpu.sync_copy(cw.at[pl.ds(0, EW)], cosw_hbm.at[pl.ds(base, EW)])


def _cosw(visn, src, dst):
    f = functools.partial(
        pl.kernel,
        out_type=jax.ShapeDtypeStruct((E,), jnp.float32),
        mesh=_mesh,
        compiler_params=pltpu.CompilerParams(needs_layout_passes=False, use_tc_tiling_on_sc=False),
        scratch_types=[
            pltpu.MemorySpace.VMEM((EW,), jnp.int32),
            pltpu.MemorySpace.VMEM((EW,), jnp.int32),
            pltpu.MemorySpace.VMEM((CH_C, VIS), jnp.bfloat16),
            pltpu.MemorySpace.VMEM((CH_C, VIS), jnp.bfloat16),
            pltpu.MemorySpace.VMEM((CH_C, VIS), jnp.bfloat16),
            pltpu.MemorySpace.VMEM((CH_C, VIS), jnp.bfloat16),
            pltpu.MemorySpace.VMEM((EW + L,), jnp.float32),
            pltpu.MemorySpace.VMEM((L, L), jnp.float32),
            pltpu.SemaphoreType.DMA,
            pltpu.SemaphoreType.DMA,
            pltpu.SemaphoreType.DMA,
            pltpu.SemaphoreType.DMA,
        ],
    )(_cosw_body)
    return f(visn, src, dst)


def _conv_body(m_hbm, src_hbm, dst_hbm, cosw_hbm, out_hbm,
               acc_sh, sidx, didx, cwv, rows0, rows1, zbuf,
               sst, sg0, sg1):
    cid = lax.axis_index("c")
    sid = lax.axis_index("s")
    wid = cid * NS + sid
    base = wid * EW

    c1 = pltpu.async_copy(src_hbm.at[pl.ds(base, EW)], sidx, sst)
    c2 = pltpu.async_copy(dst_hbm.at[pl.ds(base, EW)], didx, sst)
    c3 = pltpu.async_copy(cosw_hbm.at[pl.ds(base, EW)], cwv.at[pl.ds(0, EW)], sst)

    @pl.loop(0, ZR)
    def _zb(r):
        for k in range(D_H // L):
            zbuf[r, pl.ds(k * L, L)] = jnp.zeros((L,), jnp.float32)

    c1.wait()
    c2.wait()
    c3.wait()

    for i in range(ROWS_T // ZR):
        pltpu.async_copy(zbuf, acc_sh.at[pl.ds(sid * ROWS_T + i * ZR, ZR)], sst)
    for i in range(ROWS_T // ZR):
        pltpu.make_async_copy(zbuf, acc_sh.at[pl.ds(sid * ROWS_T, ZR)], sst).wait()
    plsc.subcore_barrier()

    def start(e0, n, buf, sem):
        pltpu.async_copy(m_hbm.at[sidx.at[pl.ds(e0, n)]],
                         buf.at[pl.ds(0, n)], sem)

    def wait(n, buf, sem):
        pltpu.make_async_copy(m_hbm.at[sidx.at[pl.ds(0, n)]],
                              buf.at[pl.ds(0, n)], sem).wait()

    def scale_scatter(e0, n, buf):
        for g in range((n + L - 1) // L):
            m = min(L, n - g * L)
            wv = cwv[pl.ds(e0 + g * L, L)]
            for jj in range(m):
                j = g * L + jj
                w = wv[jj]
                for k in range(D_H // L):
                    buf[j, pl.ds(k * L, L)] = buf[j, pl.ds(k * L, L)] * w
        pltpu.sync_copy(buf.at[pl.ds(0, n)],
                        acc_sh.at[didx.at[pl.ds(e0, n)]], add=True)

    start(0, CH_V, rows0, sg0)

    @pl.loop(0, NCH_V, step=2)
    def _chunks(c):
        e0 = c * CH_V
        start(e0 + CH_V, CH_V, rows1, sg1)
        wait(CH_V, rows0, sg0)
        scale_scatter(e0, CH_V, rows0)

        @pl.when(c + 2 < NCH_V)
        def _pref():
            start(e0 + 2 * CH_V, CH_V, rows0, sg0)

        wait(CH_V, rows1, sg1)
        scale_scatter(e0 + CH_V, CH_V, rows1)

    if REM_V:
        start(NCH_V * CH_V, REM_V, rows0, sg0)
        wait(REM_V, rows0, sg0)
        scale_scatter(NCH_V * CH_V, REM_V, rows0)

    plsc.subcore_barrier()
    pltpu.sync_copy(acc_sh.at[pl.ds(sid * ROWS_T, ROWS_T)],
                    out_hbm.at[cid, pl.ds(sid * ROWS_T, ROWS_T)])


def _conv(m, src, dst, cosw):
    f = functools.partial(
        pl.kernel,
        out_type=jax.ShapeDtypeStruct((NC, N, D_H), jnp.float32),
        mesh=_mesh,
        compiler_params=pltpu.CompilerParams(needs_layout_passes=False, use_tc_tiling_on_sc=False),
        scratch_types=[
            pltpu.MemorySpace.VMEM_SHARED((N, D_H), jnp.float32),
            pltpu.MemorySpace.VMEM((EW,), jnp.int32),
            pltpu.MemorySpace.VMEM((EW,), jnp.int32),
            pltpu.MemorySpace.VMEM((EW + L,), jnp.float32),
            pltpu.MemorySpace.VMEM((CH_V, D_H), jnp.float32),
            pltpu.MemorySpace.VMEM((CH_V, D_H), jnp.float32),
            pltpu.MemorySpace.VMEM((ZR, D_H), jnp.float32),
            pltpu.SemaphoreType.DMA,
            pltpu.SemaphoreType.DMA,
            pltpu.SemaphoreType.DMA,
        ],
    )(_conv_body)
    return f(m, src, dst, cosw)


def _tc2_body(p_ref, wc2_ref, bc2_ref, m2_ref):
    h1 = jnp.maximum(p_ref[0] + p_ref[1], 0.0)
    m2_ref[...] = jnp.dot(h1, wc2_ref[...],
                          preferred_element_type=jnp.float32) + bc2_ref[...]


def _tc2(p, W_c2, b_c2):
    return pl.pallas_call(
        _tc2_body,
        grid=(GRID,),
        in_specs=[
            pl.BlockSpec((NC, RB, D_H), lambda i: (0, i, 0)),
            pl.BlockSpec((D_H, D_H), lambda i: (0, 0)),
            pl.BlockSpec((1, D_H), lambda i: (0, 0)),
        ],
        out_specs=pl.BlockSpec((RB, D_H), lambda i: (i, 0)),
        out_shape=jax.ShapeDtypeStruct((N, D_H), jnp.float32),
    )(p, W_c2, b_c2)


def _tc3_body(p_ref, ns_ref, wcp_ref, out_ref):
    h2 = jnp.maximum(p_ref[0] + p_ref[1], 0.0)
    out_ref[...] = ns_ref[...] + jnp.sum(h2 * wcp_ref[...], axis=1)[:, None]


def _tc3(p, ns, wcp):
    return pl.pallas_call(
        _tc3_body,
        grid=(GRID,),
        in_specs=[
            pl.BlockSpec((NC, RB, D_H), lambda i: (0, i, 0)),
            pl.BlockSpec((RB, 1), lambda i: (i, 0)),
            pl.BlockSpec((1, D_H), lambda i: (0, 0)),
        ],
        out_specs=pl.BlockSpec((RB, 1), lambda i: (i, 0)),
        out_shape=jax.ShapeDtypeStruct((N, 1), jnp.float32),
    )(p, ns, wcp)


def kernel(x, edge_index, vis, W_mlp, b_mlp, W_np, b_np, W_cp, b_cp,
           W_c1, b_c1, W_c2, b_c2):
    src = edge_index[0]
    dst = edge_index[1]
    wnp = W_np.reshape(1, D_H)
    wcp = W_cp.reshape(1, D_H)
    bnp = b_np.reshape(1, 1)
    bcp = b_cp.reshape(1, 1)

    m1, visn, ns = _tc1(x, vis, W_mlp, b_mlp.reshape(1, D_H),
                        W_c1, b_c1.reshape(1, D_H), wnp, bnp, bcp)
    cosw = _cosw(visn, src, dst)
    p1 = _conv(m1, src, dst, cosw)
    m2 = _tc2(p1, W_c2, b_c2.reshape(1, D_H))
    p2 = _conv(m2, src, dst, cosw)
    out = _tc3(p2, ns, wcp)
    return out.reshape(N)


# speedup vs baseline: 2.1277x; 1.0153x over previous
import functools

import numpy as np

import jax
import jax.numpy as jnp
from jax import lax
from jax.experimental import pallas as pl
from jax.experimental.pallas import tpu as pltpu
from jax.experimental.pallas import tpu_sc as plsc

N = 10000
E = 160000
D_IN = 256
D_H = 128
VIS = 256

NC = 2
NS = 16
L = 16
NW = NC * NS
EW = E // NW
RB = 2000
GRID = N // RB

CH_C = 32
NCH_C = EW // CH_C
REM_C = EW - NCH_C * CH_C

CH_V = 64
NCH_V = EW // CH_V
REM_V = EW - NCH_V * CH_V

ROWS_T = N // NS
ZR = 25

_PERM = np.concatenate(
    [np.concatenate([np.arange(b * 32, b * 32 + 32, 2),
                     np.arange(b * 32 + 1, b * 32 + 32, 2)])
     for b in range(D_H // 32)]).astype(np.int32)

_mesh = plsc.VectorSubcoreMesh(core_axis_name="c", subcore_axis_name="s",
                               num_cores=NC, num_subcores=NS)


def _tc1_body(x_ref, vis_ref, wm_ref, bm_ref, wc1_ref, bc1_ref, wnp_ref,
              bnp_ref, bcp_ref, m1_ref, visn_ref, ns_ref):
    h = jnp.dot(x_ref[...], wm_ref[...], preferred_element_type=jnp.float32)
    h = h + bm_ref[...]
    m1_ref[...] = (jnp.dot(h, wc1_ref[...],
                           preferred_element_type=jnp.float32)
                   + bc1_ref[...]).astype(jnp.bfloat16)
    v = vis_ref[...]
    nrm = jnp.sqrt(jnp.sum(v * v, axis=1, keepdims=True)) + 1e-12
    visn_ref[...] = (v / nrm).astype(jnp.bfloat16)
    ns = jnp.sum(h * wnp_ref[...], axis=1) + bnp_ref[0, 0] + bcp_ref[0, 0]
    ns_ref[...] = ns[:, None]


def _tc1(x, vis, W_mlp, b_mlp, W_c1, b_c1, wnp, bnp, bcp):
    full = lambda s: pl.BlockSpec(s, lambda i: (0, 0))
    return pl.pallas_call(
        _tc1_body,
        grid=(GRID,),
        in_specs=[
            pl.BlockSpec((RB, D_IN), lambda i: (i, 0)),
            pl.BlockSpec((RB, VIS), lambda i: (i, 0)),
            full((D_IN, D_H)), full((1, D_H)),
            full((D_H, D_H)), full((1, D_H)),
            full((1, D_H)), full((1, 1)), full((1, 1)),
        ],
        out_specs=[
            pl.BlockSpec((RB, D_H), lambda i: (i, 0)),
            pl.BlockSpec((RB, VIS), lambda i: (i, 0)),
            pl.BlockSpec((RB, 1), lambda i: (i, 0)),
        ],
        out_shape=[
            jax.ShapeDtypeStruct((N, D_H), jnp.bfloat16),
            jax.ShapeDtypeStruct((N, VIS), jnp.bfloat16),
            jax.ShapeDtypeStruct((N, 1), jnp.float32),
        ],
    )(x, vis, W_mlp, b_mlp, W_c1, b_c1, wnp, bnp, bcp)


def _cosw_body(visn_hbm, src_hbm, dst_hbm, cosw_hbm,
               sidx, didx, srows0, drows0, srows1, drows1, cw, tbuf,
               ss0, sd0, ss1, sd1):
    wid = lax.axis_index("c") * NS + lax.axis_index("s")
    base = wid * EW
    ci = pltpu.async_copy(src_hbm.at[pl.ds(base, EW)], sidx, ss0)
    cj = pltpu.async_copy(dst_hbm.at[pl.ds(base, EW)], didx, sd0)
    ci.wait()
    cj.wait()

    lane = lax.iota(jnp.int32, L)

    def start(e0, n, sbuf, dbuf, sems, semd):
        pltpu.async_copy(visn_hbm.at[sidx.at[pl.ds(e0, n)]],
                         sbuf.at[pl.ds(0, n)], sems)
        pltpu.async_copy(visn_hbm.at[didx.at[pl.ds(e0, n)]],
                         dbuf.at[pl.ds(0, n)], semd)

    def wait(n, sbuf, dbuf, sems, semd):
        pltpu.make_async_copy(visn_hbm.at[sidx.at[pl.ds(0, n)]],
                              sbuf.at[pl.ds(0, n)], sems).wait()
        pltpu.make_async_copy(visn_hbm.at[didx.at[pl.ds(0, n)]],
                              dbuf.at[pl.ds(0, n)], semd).wait()

    def compute(e0, n, sbuf, dbuf):
        B2 = 2 * L
        NB = VIS // B2
        hmask = jnp.full((L,), jnp.int32(-65536))
        for g in range((n + L - 1) // L):
            m = min(L, n - g * L)
            for jj in range(m):
                j = g * L + jj
                accs = [None, None, None, None]
                for b in range(NB):
                    sv = sbuf[j, pl.ds(b * B2, B2)]
                    dv = dbuf[j, pl.ds(b * B2, B2)]
                    p = sv * dv
                    pi = plsc.bitcast(p, jnp.int32)
                    pe = plsc.bitcast(pi << 16, jnp.float32)
                    po = plsc.bitcast(pi & hmask, jnp.float32)
                    q = 2 * b
                    for (pp, i) in ((pe, q % 4), (po, (q + 1) % 4)):
                        accs[i] = pp if accs[i] is None else accs[i] + pp
                tbuf[jj] = (accs[0] + accs[1]) + (accs[2] + accs[3])
            rs = plsc.load_gather(tbuf, [lane, jnp.zeros((L,), jnp.int32)])
            for k in range(1, L):
                rs = rs + plsc.load_gather(tbuf, [lane, jnp.full((L,), k, jnp.int32)])
            cw[pl.ds(e0 + g * L, L)] = rs

    start(0, CH_C, srows0, drows0, ss0, sd0)

    @pl.loop(0, NCH_C, step=2)
    def _chunks(c):
        e0 = c * CH_C
        start(e0 + CH_C, CH_C, srows1, drows1, ss1, sd1)
        wait(CH_C, srows0, drows0, ss0, sd0)
        compute(e0, CH_C, srows0, drows0)

        @pl.when(c + 2 < NCH_C)
        def _pref():
            start(e0 + 2 * CH_C, CH_C, srows0, drows0, ss0, sd0)

        wait(CH_C, srows1, drows1, ss1, sd1)
        compute(e0 + CH_C, CH_C, srows1, drows1)

    if REM_C:
        start(NCH_C * CH_C, REM_C, srows0, drows0, ss0, sd0)
        wait(REM_C, srows0, drows0, ss0, sd0)
        compute(NCH_C * CH_C, REM_C, srows0, drows0)

    pltpu.sync_copy(cw.at[pl.ds(0, EW)], cosw_hbm.at[pl.ds(base, EW)])


def _cosw(visn, src, dst):
    f = functools.partial(
        pl.kernel,
        out_type=jax.ShapeDtypeStruct((E,), jnp.float32),
        mesh=_mesh,
        compiler_params=pltpu.CompilerParams(needs_layout_passes=False, use_tc_tiling_on_sc=False),
        scratch_types=[
            pltpu.MemorySpace.VMEM((EW,), jnp.int32),
            pltpu.MemorySpace.VMEM((EW,), jnp.int32),
            pltpu.MemorySpace.VMEM((CH_C, VIS), jnp.bfloat16),
            pltpu.MemorySpace.VMEM((CH_C, VIS), jnp.bfloat16),
            pltpu.MemorySpace.VMEM((CH_C, VIS), jnp.bfloat16),
            pltpu.MemorySpace.VMEM((CH_C, VIS), jnp.bfloat16),
            pltpu.MemorySpace.VMEM((EW + L,), jnp.float32),
            pltpu.MemorySpace.VMEM((L, L), jnp.float32),
            pltpu.SemaphoreType.DMA,
            pltpu.SemaphoreType.DMA,
            pltpu.SemaphoreType.DMA,
            pltpu.SemaphoreType.DMA,
        ],
    )(_cosw_body)
    return f(visn, src, dst)


def _conv_body(m_hbm, src_hbm, dst_hbm, cosw_hbm, out_hbm,
               acc_sh, sidx, didx, cwv, rows0, rows1, frows0, zbuf,
               sst, sg0, sg1):
    cid = lax.axis_index("c")
    sid = lax.axis_index("s")
    wid = cid * NS + sid
    base = wid * EW

    c1 = pltpu.async_copy(src_hbm.at[pl.ds(base, EW)], sidx, sst)
    c2 = pltpu.async_copy(dst_hbm.at[pl.ds(base, EW)], didx, sst)
    c3 = pltpu.async_copy(cosw_hbm.at[pl.ds(base, EW)], cwv.at[pl.ds(0, EW)], sst)

    @pl.loop(0, ZR)
    def _zb(r):
        for k in range(D_H // L):
            zbuf[r, pl.ds(k * L, L)] = jnp.zeros((L,), jnp.float32)

    c1.wait()
    c2.wait()
    c3.wait()

    for i in range(ROWS_T // ZR):
        pltpu.async_copy(zbuf, acc_sh.at[pl.ds(sid * ROWS_T + i * ZR, ZR)], sst)
    for i in range(ROWS_T // ZR):
        pltpu.make_async_copy(zbuf, acc_sh.at[pl.ds(sid * ROWS_T, ZR)], sst).wait()
    plsc.subcore_barrier()

    def start(e0, n, buf, sem):
        pltpu.async_copy(m_hbm.at[sidx.at[pl.ds(e0, n)]],
                         buf.at[pl.ds(0, n)], sem)

    def wait(n, buf, sem):
        pltpu.make_async_copy(m_hbm.at[sidx.at[pl.ds(0, n)]],
                              buf.at[pl.ds(0, n)], sem).wait()

    def scale_scatter(e0, n, buf, fbuf):
        B2 = 2 * L
        hmask = jnp.full((L,), jnp.int32(-65536))
        for g in range((n + L - 1) // L):
            m = min(L, n - g * L)
            wv = cwv[pl.ds(e0 + g * L, L)]
            for jj in range(m):
                j = g * L + jj
                w = wv[jj]
                for b in range(D_H // B2):
                    mv = buf[j, pl.ds(b * B2, B2)]
                    pi = plsc.bitcast(mv, jnp.int32)
                    pe = plsc.bitcast(pi << 16, jnp.float32)
                    po = plsc.bitcast(pi & hmask, jnp.float32)
                    fbuf[j, pl.ds(b * B2, L)] = pe * w
                    fbuf[j, pl.ds(b * B2 + L, L)] = po * w
        pltpu.sync_copy(fbuf.at[pl.ds(0, n)],
                        acc_sh.at[didx.at[pl.ds(e0, n)]], add=True)

    start(0, CH_V, rows0, sg0)

    @pl.loop(0, NCH_V, step=2)
    def _chunks(c):
        e0 = c * CH_V
        start(e0 + CH_V, CH_V, rows1, sg1)
        wait(CH_V, rows0, sg0)
        scale_scatter(e0, CH_V, rows0, frows0)

        @pl.when(c + 2 < NCH_V)
        def _pref():
            start(e0 + 2 * CH_V, CH_V, rows0, sg0)

        wait(CH_V, rows1, sg1)
        scale_scatter(e0 + CH_V, CH_V, rows1, frows0)

    if REM_V:
        start(NCH_V * CH_V, REM_V, rows0, sg0)
        wait(REM_V, rows0, sg0)
        scale_scatter(NCH_V * CH_V, REM_V, rows0, frows0)

    plsc.subcore_barrier()
    pltpu.sync_copy(acc_sh.at[pl.ds(sid * ROWS_T, ROWS_T)],
                    out_hbm.at[cid, pl.ds(sid * ROWS_T, ROWS_T)])


def _conv(m, src, dst, cosw):
    f = functools.partial(
        pl.kernel,
        out_type=jax.ShapeDtypeStruct((NC, N, D_H), jnp.float32),
        mesh=_mesh,
        compiler_params=pltpu.CompilerParams(needs_layout_passes=False, use_tc_tiling_on_sc=False),
        scratch_types=[
            pltpu.MemorySpace.VMEM_SHARED((N, D_H), jnp.float32),
            pltpu.MemorySpace.VMEM((EW,), jnp.int32),
            pltpu.MemorySpace.VMEM((EW,), jnp.int32),
            pltpu.MemorySpace.VMEM((EW + L,), jnp.float32),
            pltpu.MemorySpace.VMEM((CH_V, D_H), jnp.bfloat16),
            pltpu.MemorySpace.VMEM((CH_V, D_H), jnp.bfloat16),
            pltpu.MemorySpace.VMEM((CH_V, D_H), jnp.float32),
            pltpu.MemorySpace.VMEM((ZR, D_H), jnp.float32),
            pltpu.SemaphoreType.DMA,
            pltpu.SemaphoreType.DMA,
            pltpu.SemaphoreType.DMA,
        ],
    )(_conv_body)
    return f(m, src, dst, cosw)


def _tc2_body(p_ref, wc2_ref, bc2_ref, m2_ref):
    h1 = jnp.maximum(p_ref[0] + p_ref[1], 0.0)
    m2_ref[...] = (jnp.dot(h1, wc2_ref[...],
                           preferred_element_type=jnp.float32)
                   + bc2_ref[...]).astype(jnp.bfloat16)


def _tc2(p, W_c2, b_c2):
    return pl.pallas_call(
        _tc2_body,
        grid=(GRID,),
        in_specs=[
            pl.BlockSpec((NC, RB, D_H), lambda i: (0, i, 0)),
            pl.BlockSpec((D_H, D_H), lambda i: (0, 0)),
            pl.BlockSpec((1, D_H), lambda i: (0, 0)),
        ],
        out_specs=pl.BlockSpec((RB, D_H), lambda i: (i, 0)),
        out_shape=jax.ShapeDtypeStruct((N, D_H), jnp.bfloat16),
    )(p, W_c2, b_c2)


def _tc3_body(p_ref, ns_ref, wcp_ref, out_ref):
    h2 = jnp.maximum(p_ref[0] + p_ref[1], 0.0)
    out_ref[...] = ns_ref[...] + jnp.sum(h2 * wcp_ref[...], axis=1)[:, None]


def _tc3(p, ns, wcp):
    return pl.pallas_call(
        _tc3_body,
        grid=(GRID,),
        in_specs=[
            pl.BlockSpec((NC, RB, D_H), lambda i: (0, i, 0)),
            pl.BlockSpec((RB, 1), lambda i: (i, 0)),
            pl.BlockSpec((1, D_H), lambda i: (0, 0)),
        ],
        out_specs=pl.BlockSpec((RB, 1), lambda i: (i, 0)),
        out_shape=jax.ShapeDtypeStruct((N, 1), jnp.float32),
    )(p, ns, wcp)


def kernel(x, edge_index, vis, W_mlp, b_mlp, W_np, b_np, W_cp, b_cp,
           W_c1, b_c1, W_c2, b_c2):
    src = edge_index[0]
    dst = edge_index[1]
    wnp = W_np.reshape(1, D_H)
    bnp = b_np.reshape(1, 1)
    bcp = b_cp.reshape(1, 1)
    perm = jnp.asarray(_PERM)
    wc1p = W_c1[:, perm]
    bc1p = b_c1[perm]
    wc2p = W_c2[perm, :][:, perm]
    bc2p = b_c2[perm]
    wcpp = W_cp[perm, :].reshape(1, D_H)

    m1, visn, ns = _tc1(x, vis, W_mlp, b_mlp.reshape(1, D_H),
                        wc1p, bc1p.reshape(1, D_H), wnp, bnp, bcp)
    cosw = _cosw(visn, src, dst)
    p1 = _conv(m1, src, dst, cosw)
    m2 = _tc2(p1, wc2p, bc2p.reshape(1, D_H))
    p2 = _conv(m2, src, dst, cosw)
    out = _tc3(p2, ns, wcpp)
    return out.reshape(N)
